# baseline (speedup 1.0000x reference)
import functools

import jax
import jax.numpy as jnp
from jax import lax
from jax.experimental import pallas as pl
from jax.experimental.pallas import tpu as pltpu
from jax.experimental.pallas import tpu_sc as plsc

N_TOKENS = 16 * 1024
DIM = 256
K_CODES = 8192
WIN = 2736
SUB = 8

N_TILE = 256


def _bf16_rne(v):
    u = lax.bitcast_convert_type(v, jnp.uint32)
    u = (u + jnp.uint32(0x7FFF) + ((u >> 16) & jnp.uint32(1))) & jnp.uint32(0xFFFF0000)
    return lax.bitcast_convert_type(u, jnp.float32)


def _argmin_body(x_ref, e_ref, xx_ref, ee_ref, out_ref, scr_a, scr_b):
    xb = x_ref[...]
    xbb = (xb + xb).astype(jnp.bfloat16)
    xx = xx_ref[0, 0, :][None, :]
    sub_iota = lax.broadcasted_iota(jnp.int32, (SUB, N_TILE), 0).astype(jnp.float32)
    acc_v = jnp.full((N_TILE,), jnp.inf, jnp.float32)
    acc_i = jnp.full((N_TILE,), float(K_CODES), jnp.float32)
    for w in range(3):
        lo = w * WIN
        hi = min(lo + WIN, K_CODES)
        mm2 = lax.dot_general(e_ref[lo:hi, :], xbb,
                              (((1,), (1,)), ((), ())),
                              preferred_element_type=jnp.float32)
        scr = (scr_a, scr_b)[w % 2]
        scr[0:hi - lo, :] = mm2
        ntiles = (hi - lo) // SUB

        def tile_step(t, carry):
            run_v, run_t = carry
            mm2_t = scr[pl.ds(t * SUB, SUB), :]
            ee_t = ee_ref[0, pl.ds(lo + t * SUB, SUB), :]
            at = (xx - mm2_t) + ee_t
            lt = at < run_v
            tf = t.astype(jnp.float32)
            run_t = jnp.where(lt, tf, run_t)
            run_v = jnp.where(lt, at, run_v)
            return run_v, run_t

        run_v0 = jnp.full((SUB, N_TILE), jnp.inf, jnp.float32)
        run_t0 = jnp.zeros((SUB, N_TILE), jnp.float32)
        run_v, run_t = lax.fori_loop(0, ntiles, tile_step, (run_v0, run_t0),
                                     unroll=4)
        m = jnp.min(run_v, axis=0)
        kfin = (run_t * float(SUB) + sub_iota) + float(lo)
        idx = jnp.min(jnp.where(run_v == m[None, :], kfin, float(K_CODES)),
                      axis=0)
        upd = m < acc_v
        acc_i = jnp.where(upd, idx, acc_i)
        acc_v = jnp.where(upd, _bf16_rne(m), acc_v)
    out_ref[0, 0, :] = acc_i.astype(jnp.int32)


def _nearest_code(xf, embed_bf16, xx3, ee2):
    nblk = N_TOKENS // N_TILE
    ind3 = pl.pallas_call(
        _argmin_body,
        grid=(nblk,),
        in_specs=[
            pl.BlockSpec((N_TILE, DIM), lambda i: (i, 0)),
            pl.BlockSpec((K_CODES, DIM), lambda i: (0, 0)),
            pl.BlockSpec((1, 1, N_TILE), lambda i: (i, 0, 0)),
            pl.BlockSpec((1, K_CODES, 1), lambda i: (0, 0, 0)),
        ],
        out_specs=pl.BlockSpec((1, 1, N_TILE), lambda i: (i, 0, 0)),
        out_shape=jax.ShapeDtypeStruct((nblk, 1, N_TILE), jnp.int32),
        scratch_shapes=[pltpu.VMEM((WIN, N_TILE), jnp.float32),
                        pltpu.VMEM((WIN, N_TILE), jnp.float32)],
    )(xf, embed_bf16, xx3, ee2)
    return ind3.reshape(N_TOKENS)


@functools.cache
def _make_gather():
    info = plsc.get_sparse_core_info()
    nw = info.num_cores * info.num_subcores
    b_per_w = N_TOKENS // nw
    chunk = 128
    n_chunks = b_per_w // chunk
    mesh = plsc.VectorSubcoreMesh(core_axis_name="c", subcore_axis_name="s")

    @functools.partial(
        pl.kernel, mesh=mesh,
        out_type=jax.ShapeDtypeStruct((N_TOKENS, DIM), jnp.float32),
        scratch_types=[
            pltpu.VMEM((chunk,), jnp.int32),
            pltpu.VMEM((chunk, DIM), jnp.float32),
            pltpu.SemaphoreType.DMA,
        ],
    )
    def gather(table_hbm, idx_hbm, out_hbm, idx_v, rows_v, sem):
        wid = lax.axis_index("s") * info.num_cores + lax.axis_index("c")
        base = wid * b_per_w
        for j in range(n_chunks):
            off = base + j * chunk
            pltpu.sync_copy(idx_hbm.at[pl.ds(off, chunk)], idx_v)
            pltpu.async_copy(table_hbm.at[idx_v], rows_v, sem).wait()
            pltpu.sync_copy(rows_v, out_hbm.at[pl.ds(off, chunk)])

    return gather


def kernel(x, embed):
    shape = x.shape
    xf = x.reshape(-1, shape[-1])
    et = embed.T
    xx = jnp.sum(xf ** 2, axis=1, keepdims=True)
    ee = jnp.sum(et ** 2, axis=0, keepdims=True)
    xx3 = xx.reshape(N_TOKENS // N_TILE, 1, N_TILE)
    ee2 = ee.reshape(1, K_CODES, 1)
    ind = _nearest_code(xf, embed.astype(jnp.bfloat16), xx3, ee2)
    quantize = _make_gather()(embed, ind)
    return (quantize.reshape(shape), ind.reshape(shape[:-1]))


# speedup vs baseline: 6.3570x; 6.3570x over previous
import functools

import jax
import jax.numpy as jnp
from jax import lax
from jax.experimental import pallas as pl
from jax.experimental.pallas import tpu as pltpu
from jax.experimental.pallas import tpu_sc as plsc

N_TOKENS = 16 * 1024
DIM = 256
K_CODES = 8192
WIN = 2736
SUB = 8

N_TILE = 256


def _bf16_rne(v):
    u = lax.bitcast_convert_type(v, jnp.uint32)
    u = (u + jnp.uint32(0x7FFF) + ((u >> 16) & jnp.uint32(1))) & jnp.uint32(0xFFFF0000)
    return lax.bitcast_convert_type(u, jnp.float32)


def _argmin_body(x_ref, e_ref, xx_ref, ee_ref, out_ref):
    xb = x_ref[...]
    xbb = (xb + xb).astype(jnp.bfloat16)
    xx = xx_ref[0, 0, :][None, :]
    sub_iota = lax.broadcasted_iota(
        jnp.int32, (SUB, N_TILE), 0).astype(jnp.float32)
    big = float(K_CODES)
    acc_v = jnp.full((N_TILE,), jnp.inf, jnp.float32)
    acc_i = jnp.full((N_TILE,), big, jnp.float32)
    for w in range(3):
        lo = w * WIN
        hi = min(lo + WIN, K_CODES)
        nt = (hi - lo) // SUB
        mm2 = lax.dot_general(e_ref[lo:hi, :], xbb,
                              (((1,), (1,)), ((), ())),
                              preferred_element_type=jnp.float32)
        ee = ee_ref[0, lo:hi, :]
        a3 = ((xx - mm2) + ee).reshape(nt, SUB, N_TILE)
        m3 = jnp.min(a3, axis=0)
        ts = lax.broadcasted_iota(
            jnp.int32, (nt, 1, 1), 0).astype(jnp.float32)
        t3 = jnp.min(jnp.where(a3 == m3[None], ts, big), axis=0)
        m = jnp.min(m3, axis=0)
        kfin = (t3 * float(SUB) + sub_iota) + float(lo)
        idx = jnp.min(jnp.where(m3 == m[None, :], kfin, big), axis=0)
        upd = m < acc_v
        acc_i = jnp.where(upd, idx, acc_i)
        acc_v = jnp.where(upd, _bf16_rne(m), acc_v)
    out_ref[0, 0, :] = acc_i.astype(jnp.int32)


def _nearest_code(xf, embed_bf16, xx3, ee2):
    nblk = N_TOKENS // N_TILE
    ind3 = pl.pallas_call(
        _argmin_body,
        grid=(nblk,),
        in_specs=[
            pl.BlockSpec((N_TILE, DIM), lambda i: (i, 0)),
            pl.BlockSpec((K_CODES, DIM), lambda i: (0, 0)),
            pl.BlockSpec((1, 1, N_TILE), lambda i: (i, 0, 0)),
            pl.BlockSpec((1, K_CODES, 1), lambda i: (0, 0, 0)),
        ],
        out_specs=pl.BlockSpec((1, 1, N_TILE), lambda i: (i, 0, 0)),
        out_shape=jax.ShapeDtypeStruct((nblk, 1, N_TILE), jnp.int32),
    )(xf, embed_bf16, xx3, ee2)
    return ind3.reshape(N_TOKENS)


@functools.cache
def _make_gather():
    info = plsc.get_sparse_core_info()
    nw = info.num_cores * info.num_subcores
    b_per_w = N_TOKENS // nw
    chunk = 128
    n_chunks = b_per_w // chunk
    mesh = plsc.VectorSubcoreMesh(core_axis_name="c", subcore_axis_name="s")

    @functools.partial(
        pl.kernel, mesh=mesh,
        out_type=jax.ShapeDtypeStruct((N_TOKENS, DIM), jnp.float32),
        scratch_types=[
            pltpu.VMEM((chunk,), jnp.int32),
            pltpu.VMEM((chunk, DIM), jnp.float32),
            pltpu.SemaphoreType.DMA,
        ],
    )
    def gather(table_hbm, idx_hbm, out_hbm, idx_v, rows_v, sem):
        wid = lax.axis_index("s") * info.num_cores + lax.axis_index("c")
        base = wid * b_per_w
        for j in range(n_chunks):
            off = base + j * chunk
            pltpu.sync_copy(idx_hbm.at[pl.ds(off, chunk)], idx_v)
            pltpu.async_copy(table_hbm.at[idx_v], rows_v, sem).wait()
            pltpu.sync_copy(rows_v, out_hbm.at[pl.ds(off, chunk)])

    return gather


def kernel(x, embed):
    shape = x.shape
    xf = x.reshape(-1, shape[-1])
    et = embed.T
    xx = jnp.sum(xf ** 2, axis=1, keepdims=True)
    ee = jnp.sum(et ** 2, axis=0, keepdims=True)
    xx3 = xx.reshape(N_TOKENS // N_TILE, 1, N_TILE)
    ee2 = ee.reshape(1, K_CODES, 1)
    ind = _nearest_code(xf, embed.astype(jnp.bfloat16), xx3, ee2)
    quantize = _make_gather()(embed, ind)
    return (quantize.reshape(shape), ind.reshape(shape[:-1]))


# speedup vs baseline: 6.8678x; 1.0804x over previous
import functools

import jax
import jax.numpy as jnp
from jax import lax
from jax.experimental import pallas as pl
from jax.experimental.pallas import tpu as pltpu
from jax.experimental.pallas import tpu_sc as plsc

N_TOKENS = 16 * 1024
DIM = 256
K_CODES = 8192
WIN = 2736
SUB = 8

N_TILE = 512


def _bf16_rne(v):
    u = lax.bitcast_convert_type(v, jnp.uint32)
    u = (u + jnp.uint32(0x7FFF) + ((u >> 16) & jnp.uint32(1))) & jnp.uint32(0xFFFF0000)
    return lax.bitcast_convert_type(u, jnp.float32)


def _argmin_body(x_ref, e_ref, xx_ref, ee_ref, out_ref):
    xb = x_ref[...]
    xbb = (xb + xb).astype(jnp.bfloat16)
    xx = xx_ref[0, 0, :][None, :]
    sub_iota = lax.broadcasted_iota(
        jnp.int32, (SUB, N_TILE), 0).astype(jnp.float32)
    big = float(K_CODES)
    acc_v = jnp.full((N_TILE,), jnp.inf, jnp.float32)
    acc_i = jnp.full((N_TILE,), big, jnp.float32)
    for w in range(3):
        lo = w * WIN
        hi = min(lo + WIN, K_CODES)
        nt = (hi - lo) // SUB
        mm2 = lax.dot_general(e_ref[lo:hi, :], xbb,
                              (((1,), (1,)), ((), ())),
                              preferred_element_type=jnp.float32)
        ee = ee_ref[0, lo:hi, :]
        a3 = ((xx - mm2) + ee).reshape(nt, SUB, N_TILE)
        m3 = jnp.min(a3, axis=0)
        ts = lax.broadcasted_iota(
            jnp.int32, (nt, 1, 1), 0).astype(jnp.float32)
        t3 = jnp.min(jnp.where(a3 == m3[None], ts, big), axis=0)
        m = jnp.min(m3, axis=0)
        kfin = (t3 * float(SUB) + sub_iota) + float(lo)
        idx = jnp.min(jnp.where(m3 == m[None, :], kfin, big), axis=0)
        upd = m < acc_v
        acc_i = jnp.where(upd, idx, acc_i)
        acc_v = jnp.where(upd, _bf16_rne(m), acc_v)
    out_ref[0, 0, :] = acc_i.astype(jnp.int32)


def _nearest_code(xf, embed_bf16, xx3, ee2):
    nblk = N_TOKENS // N_TILE
    ind3 = pl.pallas_call(
        _argmin_body,
        grid=(nblk,),
        in_specs=[
            pl.BlockSpec((N_TILE, DIM), lambda i: (i, 0)),
            pl.BlockSpec((K_CODES, DIM), lambda i: (0, 0)),
            pl.BlockSpec((1, 1, N_TILE), lambda i: (i, 0, 0)),
            pl.BlockSpec((1, K_CODES, 1), lambda i: (0, 0, 0)),
        ],
        out_specs=pl.BlockSpec((1, 1, N_TILE), lambda i: (i, 0, 0)),
        out_shape=jax.ShapeDtypeStruct((nblk, 1, N_TILE), jnp.int32),
    )(xf, embed_bf16, xx3, ee2)
    return ind3.reshape(N_TOKENS)


@functools.cache
def _make_gather():
    info = plsc.get_sparse_core_info()
    nw = info.num_cores * info.num_subcores
    b_per_w = N_TOKENS // nw
    chunk = 128
    n_chunks = b_per_w // chunk
    mesh = plsc.VectorSubcoreMesh(core_axis_name="c", subcore_axis_name="s")

    @functools.partial(
        pl.kernel, mesh=mesh,
        out_type=jax.ShapeDtypeStruct((N_TOKENS, DIM), jnp.float32),
        scratch_types=[
            pltpu.VMEM((chunk,), jnp.int32),
            pltpu.VMEM((chunk, DIM), jnp.float32),
            pltpu.SemaphoreType.DMA,
        ],
    )
    def gather(table_hbm, idx_hbm, out_hbm, idx_v, rows_v, sem):
        wid = lax.axis_index("s") * info.num_cores + lax.axis_index("c")
        base = wid * b_per_w
        for j in range(n_chunks):
            off = base + j * chunk
            pltpu.sync_copy(idx_hbm.at[pl.ds(off, chunk)], idx_v)
            pltpu.async_copy(table_hbm.at[idx_v], rows_v, sem).wait()
            pltpu.sync_copy(rows_v, out_hbm.at[pl.ds(off, chunk)])

    return gather


def kernel(x, embed):
    shape = x.shape
    xf = x.reshape(-1, shape[-1])
    et = embed.T
    xx = jnp.sum(xf ** 2, axis=1, keepdims=True)
    ee = jnp.sum(et ** 2, axis=0, keepdims=True)
    xx3 = xx.reshape(N_TOKENS // N_TILE, 1, N_TILE)
    ee2 = ee.reshape(1, K_CODES, 1)
    ind = _nearest_code(xf, embed.astype(jnp.bfloat16), xx3, ee2)
    quantize = _make_gather()(embed, ind)
    return (quantize.reshape(shape), ind.reshape(shape[:-1]))


# speedup vs baseline: 6.9916x; 1.0180x over previous
import functools

import jax
import jax.numpy as jnp
from jax import lax
from jax.experimental import pallas as pl
from jax.experimental.pallas import tpu as pltpu
from jax.experimental.pallas import tpu_sc as plsc

N_TOKENS = 16 * 1024
DIM = 256
K_CODES = 8192
WIN = 2736
SUB = 8

N_TILE = 1024


def _bf16_rne(v):
    u = lax.bitcast_convert_type(v, jnp.uint32)
    u = (u + jnp.uint32(0x7FFF) + ((u >> 16) & jnp.uint32(1))) & jnp.uint32(0xFFFF0000)
    return lax.bitcast_convert_type(u, jnp.float32)


def _argmin_body(x_ref, e_ref, xx_ref, ee_ref, out_ref):
    xb = x_ref[...]
    xbb = (xb + xb).astype(jnp.bfloat16)
    xx = xx_ref[0, 0, :][None, :]
    sub_iota = lax.broadcasted_iota(
        jnp.int32, (SUB, N_TILE), 0).astype(jnp.float32)
    big = float(K_CODES)
    acc_v = jnp.full((N_TILE,), jnp.inf, jnp.float32)
    acc_i = jnp.full((N_TILE,), big, jnp.float32)
    for w in range(3):
        lo = w * WIN
        hi = min(lo + WIN, K_CODES)
        nt = (hi - lo) // SUB
        mm2 = lax.dot_general(e_ref[lo:hi, :], xbb,
                              (((1,), (1,)), ((), ())),
                              preferred_element_type=jnp.float32)
        ee = ee_ref[0, lo:hi, :]
        a3 = ((xx - mm2) + ee).reshape(nt, SUB, N_TILE)
        m3 = jnp.min(a3, axis=0)
        ts = lax.broadcasted_iota(
            jnp.int32, (nt, 1, 1), 0).astype(jnp.float32)
        t3 = jnp.min(jnp.where(a3 == m3[None], ts, big), axis=0)
        m = jnp.min(m3, axis=0)
        kfin = (t3 * float(SUB) + sub_iota) + float(lo)
        idx = jnp.min(jnp.where(m3 == m[None, :], kfin, big), axis=0)
        upd = m < acc_v
        acc_i = jnp.where(upd, idx, acc_i)
        acc_v = jnp.where(upd, _bf16_rne(m), acc_v)
    out_ref[0, 0, :] = acc_i.astype(jnp.int32)


def _nearest_code(xf, embed_bf16, xx3, ee2):
    nblk = N_TOKENS // N_TILE
    ind3 = pl.pallas_call(
        _argmin_body,
        grid=(nblk,),
        in_specs=[
            pl.BlockSpec((N_TILE, DIM), lambda i: (i, 0)),
            pl.BlockSpec((K_CODES, DIM), lambda i: (0, 0)),
            pl.BlockSpec((1, 1, N_TILE), lambda i: (i, 0, 0)),
            pl.BlockSpec((1, K_CODES, 1), lambda i: (0, 0, 0)),
        ],
        out_specs=pl.BlockSpec((1, 1, N_TILE), lambda i: (i, 0, 0)),
        out_shape=jax.ShapeDtypeStruct((nblk, 1, N_TILE), jnp.int32),
    )(xf, embed_bf16, xx3, ee2)
    return ind3.reshape(N_TOKENS)


@functools.cache
def _make_gather():
    info = plsc.get_sparse_core_info()
    nw = info.num_cores * info.num_subcores
    b_per_w = N_TOKENS // nw
    chunk = 128
    n_chunks = b_per_w // chunk
    mesh = plsc.VectorSubcoreMesh(core_axis_name="c", subcore_axis_name="s")

    @functools.partial(
        pl.kernel, mesh=mesh,
        out_type=jax.ShapeDtypeStruct((N_TOKENS, DIM), jnp.float32),
        scratch_types=[
            pltpu.VMEM((chunk,), jnp.int32),
            pltpu.VMEM((chunk, DIM), jnp.float32),
            pltpu.SemaphoreType.DMA,
        ],
    )
    def gather(table_hbm, idx_hbm, out_hbm, idx_v, rows_v, sem):
        wid = lax.axis_index("s") * info.num_cores + lax.axis_index("c")
        base = wid * b_per_w
        for j in range(n_chunks):
            off = base + j * chunk
            pltpu.sync_copy(idx_hbm.at[pl.ds(off, chunk)], idx_v)
            pltpu.async_copy(table_hbm.at[idx_v], rows_v, sem).wait()
            pltpu.sync_copy(rows_v, out_hbm.at[pl.ds(off, chunk)])

    return gather


def kernel(x, embed):
    shape = x.shape
    xf = x.reshape(-1, shape[-1])
    et = embed.T
    xx = jnp.sum(xf ** 2, axis=1, keepdims=True)
    ee = jnp.sum(et ** 2, axis=0, keepdims=True)
    xx3 = xx.reshape(N_TOKENS // N_TILE, 1, N_TILE)
    ee2 = ee.reshape(1, K_CODES, 1)
    ind = _nearest_code(xf, embed.astype(jnp.bfloat16), xx3, ee2)
    quantize = _make_gather()(embed, ind)
    return (quantize.reshape(shape), ind.reshape(shape[:-1]))
